# Initial kernel scaffold; baseline (speedup 1.0000x reference)
import jax
import jax.numpy as jnp
from jax.experimental import pallas as pl


def kernel(x, edge_index, batch, W, b):
    raise NotImplementedError("write your pallas kernel here")


# speedup vs baseline: 36.0199x; 36.0199x over previous
import functools

import jax
import jax.numpy as jnp
from jax import lax
from jax.experimental import pallas as pl
from jax.experimental.pallas import tpu as pltpu
from jax.experimental.pallas import tpu_sc as plsc

N = 10000
C = 128
G = 8
RATIO = 0.8
E = 320000

LANE = 128
ROWS = 79
NPAD = ROWS * LANE
DUMP = N

NW = 32
EPW = E // NW
CH = 128
NCH = -(-EPW // CH)
EPWP = NCH * CH

@functools.cache
def _mesh():
    return plsc.VectorSubcoreMesh(
        core_axis_name="c", subcore_axis_name="s", num_cores=2,
        num_subcores=16)


def _sc_deg_body(col_hbm, zeros_hbm, out_hbm, colv, ones_v, acc_sh):
    cid = lax.axis_index("c")
    sid = lax.axis_index("s")
    wid = cid * 16 + sid
    for i in range(CH // 16):
        ones_v[pl.ds(i * 16, 16)] = jnp.full((16,), 1.0, jnp.float32)

    @pl.when(sid == 0)
    def _():
        pltpu.sync_copy(zeros_hbm, acc_sh)

    pltpu.sync_copy(col_hbm.at[wid], colv)
    plsc.subcore_barrier()

    def body(j, carry):
        pltpu.sync_copy(ones_v, acc_sh.at[colv.at[j]], add=True)
        return carry

    lax.fori_loop(0, NCH, body, 0)
    plsc.subcore_barrier()

    @pl.when(sid == 0)
    def _():
        pltpu.sync_copy(acc_sh, out_hbm.at[cid])


@functools.cache
def _sc_deg_call():
    return pl.kernel(
        _sc_deg_body,
        out_type=jax.ShapeDtypeStruct((2, NPAD), jnp.float32),
        mesh=_mesh(),
        scratch_types=[
            pltpu.VMEM((NCH, CH), jnp.int32),
            pltpu.VMEM((CH,), jnp.float32),
            pltpu.VMEM_SHARED((NPAD,), jnp.float32),
        ],
    )


def _sc_deg(col3, zeros_n):
    return _sc_deg_call()(col3, zeros_n)


def _sc_agg_body(row_hbm, col_hbm, u_hbm, zeros_hbm, out_hbm,
                 rowv, colv, vals, u_sh, acc_sh):
    cid = lax.axis_index("c")
    sid = lax.axis_index("s")
    wid = cid * 16 + sid

    @pl.when(sid == 0)
    def _():
        pltpu.sync_copy(u_hbm, u_sh)

    @pl.when(sid == 1)
    def _():
        pltpu.sync_copy(zeros_hbm, acc_sh)

    pltpu.sync_copy(row_hbm.at[wid], rowv)
    pltpu.sync_copy(col_hbm.at[wid], colv)
    plsc.subcore_barrier()

    def body(j, carry):
        pltpu.sync_copy(u_sh.at[rowv.at[j]], vals)
        pltpu.sync_copy(vals, acc_sh.at[colv.at[j]], add=True)
        return carry

    lax.fori_loop(0, NCH, body, 0)
    plsc.subcore_barrier()

    @pl.when(sid == 0)
    def _():
        pltpu.sync_copy(acc_sh, out_hbm.at[cid])


@functools.cache
def _sc_agg_call():
    return pl.kernel(
        _sc_agg_body,
        out_type=jax.ShapeDtypeStruct((2, NPAD), jnp.float32),
        mesh=_mesh(),
        scratch_types=[
            pltpu.VMEM((NCH, CH), jnp.int32),
            pltpu.VMEM((NCH, CH), jnp.int32),
            pltpu.VMEM((CH,), jnp.float32),
            pltpu.VMEM_SHARED((NPAD,), jnp.float32),
            pltpu.VMEM_SHARED((NPAD,), jnp.float32),
        ],
    )


def _sc_agg(row3, col3, u, zeros_n):
    return _sc_agg_call()(row3, col3, u, zeros_n)


def _matvec_body(x_ref, w_ref, h_ref):
    h_ref[...] = jnp.dot(x_ref[...], w_ref[...],
                         preferred_element_type=jnp.float32)


_tc_matvec = pl.pallas_call(
    _matvec_body,
    grid=(ROWS,),
    in_specs=[
        pl.BlockSpec((LANE, C), lambda j: (j, 0)),
        pl.BlockSpec((C, 1), lambda j: (0, 0)),
    ],
    out_specs=pl.BlockSpec((LANE, 1), lambda j: (j, 0)),
    out_shape=jax.ShapeDtypeStruct((NPAD, 1), jnp.float32),
)


def _elem_body(degp_ref, h_ref, u_ref, dinv_ref):
    deg = degp_ref[0] + degp_ref[1] + 1.0
    dinv = 1.0 / jnp.sqrt(deg)
    u_ref[...] = h_ref[...] * dinv
    dinv_ref[...] = dinv


_tc_elem = pl.pallas_call(
    _elem_body,
    out_shape=(
        jax.ShapeDtypeStruct((ROWS, LANE), jnp.float32),
        jax.ShapeDtypeStruct((ROWS, LANE), jnp.float32),
    ),
)


def _sel_body(aggp_ref, u_ref, dinv_ref, batch_ref, b_ref,
              score_ref, wsel_ref, selm_ref):
    u = u_ref[...]
    dinv = dinv_ref[...]
    agg = aggp_ref[0] + aggp_ref[1]
    score = dinv * (agg + u) + b_ref[0, 0]
    score_ref[...] = score
    batch = batch_ref[...]

    MINI = jnp.int32(-2147483648)
    bits = lax.bitcast_convert_type(score, jnp.int32)
    ubits = jnp.where(bits >= 0, bits ^ MINI, ~bits)
    s_signed = ubits ^ MINI
    w = jnp.tanh(score)

    ii = lax.broadcasted_iota(jnp.int32, (LANE, LANE), 0)
    jj = lax.broadcasted_iota(jnp.int32, (LANE, LANE), 1)
    upper_incl = (ii <= jj).astype(jnp.float32)
    ri = lax.broadcasted_iota(jnp.int32, (ROWS, ROWS), 0)
    rj = lax.broadcasted_iota(jnp.int32, (ROWS, ROWS), 1)
    lower_strict = (rj < ri).astype(jnp.float32)

    wsel = jnp.zeros((ROWS, LANE), jnp.float32)
    selm = jnp.zeros((ROWS, LANE), jnp.float32)
    for g in range(G):
        ing = batch == g
        n_g = jnp.sum(jnp.where(ing, 1.0, 0.0))
        k_g = jnp.ceil(jnp.float32(RATIO) * n_g)

        def step(i, carry, ing=ing, k_g=k_g):
            P, a = carry
            bpos = 31 - i
            hi = lax.shift_right_logical(P, bpos) | 1
            eq = lax.shift_right_logical(ubits, bpos) == hi
            c1 = jnp.sum(jnp.where(eq & ing, 1.0, 0.0))
            take = (a + c1) >= k_g
            P2 = jnp.where(take, P | lax.shift_left(jnp.int32(1), bpos), P)
            a2 = jnp.where(take, a, a + c1)
            return (P2, a2)

        P, a = lax.fori_loop(0, 32, step, (jnp.int32(0), jnp.float32(0.0)))
        m_g = k_g - a
        strict = (s_signed > (P ^ MINI)) & ing
        ties = (ubits == P) & ing
        t = jnp.where(ties, 1.0, 0.0)
        incl = jnp.dot(t, upper_incl, preferred_element_type=jnp.float32)
        srow = incl[:, LANE - 1:LANE]
        rowpre = jnp.dot(lower_strict, srow, preferred_element_type=jnp.float32)
        excl = rowpre + incl - t
        sel = strict | (ties & (excl < m_g))
        wsel = wsel + jnp.where(sel, w, 0.0)
        selm = selm + jnp.where(sel, 1.0, 0.0)
    wsel_ref[...] = wsel
    selm_ref[...] = selm


_tc_sel = pl.pallas_call(
    _sel_body,
    out_shape=(
        jax.ShapeDtypeStruct((ROWS, LANE), jnp.float32),
        jax.ShapeDtypeStruct((ROWS, LANE), jnp.float32),
        jax.ShapeDtypeStruct((ROWS, LANE), jnp.float32),
    ),
)


def _pool_body(x_ref, wsel_ref, selm_ref, batch_ref, pooled_ref,
               vmax, vsum, vcnt):
    j = pl.program_id(0)
    neg = jnp.float32(-jnp.inf)
    ii = lax.broadcasted_iota(jnp.int32, (LANE, LANE), 0)
    jj = lax.broadcasted_iota(jnp.int32, (LANE, LANE), 1)
    eye = jnp.where(ii == jj, 1.0, 0.0)
    ones_col = jnp.ones((LANE, 1), jnp.float32)

    @pl.when(j == 0)
    def _():
        vmax[...] = jnp.full((G, LANE), neg, jnp.float32)
        vsum[...] = jnp.zeros((G, LANE), jnp.float32)
        vcnt[...] = jnp.zeros((G, LANE), jnp.float32)

    def tocol(row):
        m = eye * jnp.broadcast_to(row, (LANE, LANE))
        return jnp.dot(m, ones_col, preferred_element_type=jnp.float32)

    w_col = tocol(wsel_ref[0])
    s_col = tocol(selm_ref[0])
    b_col = tocol(batch_ref[0].astype(jnp.float32))
    val = x_ref[...] * w_col
    for g in range(G):
        m = (s_col > 0.5) & (jnp.abs(b_col - g) < 0.5)
        gmax = jnp.max(jnp.where(m, val, neg), axis=0, keepdims=True)
        gsum = jnp.sum(jnp.where(m, val, 0.0), axis=0, keepdims=True)
        gcnt = jnp.sum(jnp.where(m, 1.0, 0.0))
        vmax[g:g + 1, :] = jnp.maximum(vmax[g:g + 1, :], gmax)
        vsum[g:g + 1, :] = vsum[g:g + 1, :] + gsum
        vcnt[g:g + 1, :] = vcnt[g:g + 1, :] + gcnt

    @pl.when(j == ROWS - 1)
    def _():
        pooled_ref[:, 0:LANE] = vmax[...]
        pooled_ref[:, LANE:2 * LANE] = (vsum[...]
                                        / jnp.maximum(vcnt[...], 1.0))


_tc_pool = pl.pallas_call(
    _pool_body,
    grid=(ROWS,),
    in_specs=[
        pl.BlockSpec((LANE, C), lambda j: (j, 0)),
        pl.BlockSpec((1, 1, LANE), lambda j: (j, 0, 0)),
        pl.BlockSpec((1, 1, LANE), lambda j: (j, 0, 0)),
        pl.BlockSpec((1, 1, LANE), lambda j: (j, 0, 0)),
    ],
    out_specs=pl.BlockSpec((G, 2 * C), lambda j: (0, 0)),
    out_shape=jax.ShapeDtypeStruct((G, 2 * C), jnp.float32),
    scratch_shapes=[
        pltpu.VMEM((G, LANE), jnp.float32),
        pltpu.VMEM((G, LANE), jnp.float32),
        pltpu.VMEM((G, LANE), jnp.float32),
    ],
)


def _shard_edges(e):
    e = e.reshape(NW, EPW)
    e = jnp.pad(e, ((0, 0), (0, EPWP - EPW)), constant_values=DUMP)
    return e.reshape(NW, NCH, CH)


def kernel(x, edge_index, batch, W, b):
    row3 = _shard_edges(edge_index[0])
    col3 = _shard_edges(edge_index[1])
    x_pad = jnp.pad(x, ((0, NPAD - N), (0, 0)))
    batch_pad = jnp.pad(batch, (0, NPAD - N), constant_values=G)
    zeros_n = jnp.zeros((NPAD,), jnp.float32)

    degp = _sc_deg(col3, zeros_n)
    h = _tc_matvec(x_pad, W)
    u79, dinv79 = _tc_elem(degp.reshape(2, ROWS, LANE),
                           h.reshape(ROWS, LANE))
    aggp = _sc_agg(row3, col3, u79.reshape(NPAD), zeros_n)
    score79, wsel79, selm79 = _tc_sel(
        aggp.reshape(2, ROWS, LANE), u79, dinv79,
        batch_pad.reshape(ROWS, LANE), b.reshape(1, 1))
    pooled = _tc_pool(x_pad, wsel79.reshape(ROWS, 1, LANE),
                      selm79.reshape(ROWS, 1, LANE),
                      batch_pad.reshape(ROWS, 1, LANE))
    score = score79.reshape(NPAD)[:N]
    return (pooled, score)


# speedup vs baseline: 38.2915x; 1.0631x over previous
import functools

import jax
import jax.numpy as jnp
from jax import lax
from jax.experimental import pallas as pl
from jax.experimental.pallas import tpu as pltpu
from jax.experimental.pallas import tpu_sc as plsc

N = 10000
C = 128
G = 8
RATIO = 0.8
E = 320000

LANE = 128
ROWS = 79
NPAD = ROWS * LANE
DUMP = N

NW = 32
EPW = E // NW
CH = 128
NCH = -(-EPW // CH)
EPWP = NCH * CH

@functools.cache
def _mesh():
    return plsc.VectorSubcoreMesh(
        core_axis_name="c", subcore_axis_name="s", num_cores=2,
        num_subcores=16)


def _sc_deg_body(col_hbm, ones_hbm, zeros_hbm, out_hbm, colv, ones_v, acc_sh):
    cid = lax.axis_index("c")
    sid = lax.axis_index("s")
    wid = cid * 16 + sid

    @pl.when(sid == 0)
    def _():
        pltpu.sync_copy(zeros_hbm, acc_sh)

    pltpu.sync_copy(col_hbm.at[wid], colv)
    pltpu.sync_copy(ones_hbm, ones_v)
    plsc.subcore_barrier()
    pltpu.sync_copy(ones_v, acc_sh.at[colv], add=True)
    plsc.subcore_barrier()

    @pl.when(sid == 0)
    def _():
        pltpu.sync_copy(acc_sh, out_hbm.at[cid])


@functools.cache
def _sc_deg_call():
    return pl.kernel(
        _sc_deg_body,
        out_type=jax.ShapeDtypeStruct((2, NPAD), jnp.float32),
        mesh=_mesh(),
        scratch_types=[
            pltpu.VMEM((EPWP,), jnp.int32),
            pltpu.VMEM((EPWP,), jnp.float32),
            pltpu.VMEM_SHARED((NPAD,), jnp.float32),
        ],
    )


def _sc_deg(col3, ones_e, zeros_n):
    return _sc_deg_call()(col3, ones_e, zeros_n)


def _sc_agg_body(row_hbm, col_hbm, u_hbm, zeros_hbm, out_hbm,
                 rowv, colv, vals, u_sh, acc_sh):
    cid = lax.axis_index("c")
    sid = lax.axis_index("s")
    wid = cid * 16 + sid

    @pl.when(sid == 0)
    def _():
        pltpu.sync_copy(u_hbm, u_sh)

    @pl.when(sid == 1)
    def _():
        pltpu.sync_copy(zeros_hbm, acc_sh)

    pltpu.sync_copy(row_hbm.at[wid], rowv)
    pltpu.sync_copy(col_hbm.at[wid], colv)
    plsc.subcore_barrier()
    pltpu.sync_copy(u_sh.at[rowv], vals)
    pltpu.sync_copy(vals, acc_sh.at[colv], add=True)
    plsc.subcore_barrier()

    @pl.when(sid == 0)
    def _():
        pltpu.sync_copy(acc_sh, out_hbm.at[cid])


@functools.cache
def _sc_agg_call():
    return pl.kernel(
        _sc_agg_body,
        out_type=jax.ShapeDtypeStruct((2, NPAD), jnp.float32),
        mesh=_mesh(),
        scratch_types=[
            pltpu.VMEM((EPWP,), jnp.int32),
            pltpu.VMEM((EPWP,), jnp.int32),
            pltpu.VMEM((EPWP,), jnp.float32),
            pltpu.VMEM_SHARED((NPAD,), jnp.float32),
            pltpu.VMEM_SHARED((NPAD,), jnp.float32),
        ],
    )


def _sc_agg(row3, col3, u, zeros_n):
    return _sc_agg_call()(row3, col3, u, zeros_n)


def _matvec_body(x_ref, w_ref, h_ref):
    h_ref[...] = jnp.dot(x_ref[...], w_ref[...],
                         preferred_element_type=jnp.float32)


_tc_matvec = pl.pallas_call(
    _matvec_body,
    grid=(ROWS,),
    in_specs=[
        pl.BlockSpec((LANE, C), lambda j: (j, 0)),
        pl.BlockSpec((C, 1), lambda j: (0, 0)),
    ],
    out_specs=pl.BlockSpec((LANE, 1), lambda j: (j, 0)),
    out_shape=jax.ShapeDtypeStruct((NPAD, 1), jnp.float32),
)


def _elem_body(degp_ref, h_ref, u_ref, dinv_ref):
    deg = degp_ref[0] + degp_ref[1] + 1.0
    dinv = 1.0 / jnp.sqrt(deg)
    u_ref[...] = h_ref[...] * dinv
    dinv_ref[...] = dinv


_tc_elem = pl.pallas_call(
    _elem_body,
    out_shape=(
        jax.ShapeDtypeStruct((ROWS, LANE), jnp.float32),
        jax.ShapeDtypeStruct((ROWS, LANE), jnp.float32),
    ),
)


def _sel_body(aggp_ref, u_ref, dinv_ref, batch_ref, b_ref,
              score_ref, wsel_ref, selm_ref):
    u = u_ref[...]
    dinv = dinv_ref[...]
    agg = aggp_ref[0] + aggp_ref[1]
    score = dinv * (agg + u) + b_ref[0, 0]
    score_ref[...] = score
    batch = batch_ref[...]

    MINI = jnp.int32(-2147483648)
    bits = lax.bitcast_convert_type(score, jnp.int32)
    ubits = jnp.where(bits >= 0, bits ^ MINI, ~bits)
    s_signed = ubits ^ MINI
    w = jnp.tanh(score)

    ii = lax.broadcasted_iota(jnp.int32, (LANE, LANE), 0)
    jj = lax.broadcasted_iota(jnp.int32, (LANE, LANE), 1)
    upper_incl = (ii <= jj).astype(jnp.float32)
    ri = lax.broadcasted_iota(jnp.int32, (ROWS, ROWS), 0)
    rj = lax.broadcasted_iota(jnp.int32, (ROWS, ROWS), 1)
    lower_strict = (rj < ri).astype(jnp.float32)

    wsel = jnp.zeros((ROWS, LANE), jnp.float32)
    selm = jnp.zeros((ROWS, LANE), jnp.float32)
    for g in range(G):
        ing = batch == g
        n_g = jnp.sum(jnp.where(ing, 1.0, 0.0))
        k_g = jnp.ceil(jnp.float32(RATIO) * n_g)

        def step(i, carry, ing=ing, k_g=k_g):
            P, a = carry
            bpos = 31 - i
            hi = lax.shift_right_logical(P, bpos) | 1
            eq = lax.shift_right_logical(ubits, bpos) == hi
            c1 = jnp.sum(jnp.where(eq & ing, 1.0, 0.0))
            take = (a + c1) >= k_g
            P2 = jnp.where(take, P | lax.shift_left(jnp.int32(1), bpos), P)
            a2 = jnp.where(take, a, a + c1)
            return (P2, a2)

        P, a = lax.fori_loop(0, 32, step, (jnp.int32(0), jnp.float32(0.0)))
        m_g = k_g - a
        strict = (s_signed > (P ^ MINI)) & ing
        ties = (ubits == P) & ing
        t = jnp.where(ties, 1.0, 0.0)
        incl = jnp.dot(t, upper_incl, preferred_element_type=jnp.float32)
        srow = incl[:, LANE - 1:LANE]
        rowpre = jnp.dot(lower_strict, srow, preferred_element_type=jnp.float32)
        excl = rowpre + incl - t
        sel = strict | (ties & (excl < m_g))
        wsel = wsel + jnp.where(sel, w, 0.0)
        selm = selm + jnp.where(sel, 1.0, 0.0)
    wsel_ref[...] = wsel
    selm_ref[...] = selm


_tc_sel = pl.pallas_call(
    _sel_body,
    out_shape=(
        jax.ShapeDtypeStruct((ROWS, LANE), jnp.float32),
        jax.ShapeDtypeStruct((ROWS, LANE), jnp.float32),
        jax.ShapeDtypeStruct((ROWS, LANE), jnp.float32),
    ),
)


def _pool_body(x_ref, wsel_ref, selm_ref, batch_ref, pooled_ref,
               vmax, vsum, vcnt):
    j = pl.program_id(0)
    neg = jnp.float32(-jnp.inf)
    ii = lax.broadcasted_iota(jnp.int32, (LANE, LANE), 0)
    jj = lax.broadcasted_iota(jnp.int32, (LANE, LANE), 1)
    eye = jnp.where(ii == jj, 1.0, 0.0)
    ones_col = jnp.ones((LANE, 1), jnp.float32)

    @pl.when(j == 0)
    def _():
        vmax[...] = jnp.full((G, LANE), neg, jnp.float32)
        vsum[...] = jnp.zeros((G, LANE), jnp.float32)
        vcnt[...] = jnp.zeros((G, LANE), jnp.float32)

    def tocol(row):
        m = eye * jnp.broadcast_to(row, (LANE, LANE))
        return jnp.dot(m, ones_col, preferred_element_type=jnp.float32)

    w_col = tocol(wsel_ref[0])
    s_col = tocol(selm_ref[0])
    b_col = tocol(batch_ref[0].astype(jnp.float32))
    val = x_ref[...] * w_col
    for g in range(G):
        m = (s_col > 0.5) & (jnp.abs(b_col - g) < 0.5)
        gmax = jnp.max(jnp.where(m, val, neg), axis=0, keepdims=True)
        gsum = jnp.sum(jnp.where(m, val, 0.0), axis=0, keepdims=True)
        gcnt = jnp.sum(jnp.where(m, 1.0, 0.0))
        vmax[g:g + 1, :] = jnp.maximum(vmax[g:g + 1, :], gmax)
        vsum[g:g + 1, :] = vsum[g:g + 1, :] + gsum
        vcnt[g:g + 1, :] = vcnt[g:g + 1, :] + gcnt

    @pl.when(j == ROWS - 1)
    def _():
        pooled_ref[:, 0:LANE] = vmax[...]
        pooled_ref[:, LANE:2 * LANE] = (vsum[...]
                                        / jnp.maximum(vcnt[...], 1.0))


_tc_pool = pl.pallas_call(
    _pool_body,
    grid=(ROWS,),
    in_specs=[
        pl.BlockSpec((LANE, C), lambda j: (j, 0)),
        pl.BlockSpec((1, 1, LANE), lambda j: (j, 0, 0)),
        pl.BlockSpec((1, 1, LANE), lambda j: (j, 0, 0)),
        pl.BlockSpec((1, 1, LANE), lambda j: (j, 0, 0)),
    ],
    out_specs=pl.BlockSpec((G, 2 * C), lambda j: (0, 0)),
    out_shape=jax.ShapeDtypeStruct((G, 2 * C), jnp.float32),
    scratch_shapes=[
        pltpu.VMEM((G, LANE), jnp.float32),
        pltpu.VMEM((G, LANE), jnp.float32),
        pltpu.VMEM((G, LANE), jnp.float32),
    ],
)


def _shard_edges(e):
    e = e.reshape(NW, EPW)
    return jnp.pad(e, ((0, 0), (0, EPWP - EPW)), constant_values=DUMP)


def kernel(x, edge_index, batch, W, b):
    row3 = _shard_edges(edge_index[0])
    col3 = _shard_edges(edge_index[1])
    x_pad = jnp.pad(x, ((0, NPAD - N), (0, 0)))
    batch_pad = jnp.pad(batch, (0, NPAD - N), constant_values=G)
    zeros_n = jnp.zeros((NPAD,), jnp.float32)
    ones_e = jnp.ones((EPWP,), jnp.float32)

    degp = _sc_deg(col3, ones_e, zeros_n)
    h = _tc_matvec(x_pad, W)
    u79, dinv79 = _tc_elem(degp.reshape(2, ROWS, LANE),
                           h.reshape(ROWS, LANE))
    aggp = _sc_agg(row3, col3, u79.reshape(NPAD), zeros_n)
    score79, wsel79, selm79 = _tc_sel(
        aggp.reshape(2, ROWS, LANE), u79, dinv79,
        batch_pad.reshape(ROWS, LANE), b.reshape(1, 1))
    pooled = _tc_pool(x_pad, wsel79.reshape(ROWS, 1, LANE),
                      selm79.reshape(ROWS, 1, LANE),
                      batch_pad.reshape(ROWS, 1, LANE))
    score = score79.reshape(NPAD)[:N]
    return (pooled, score)


# speedup vs baseline: 39.2922x; 1.0261x over previous
import functools

import jax
import jax.numpy as jnp
from jax import lax
from jax.experimental import pallas as pl
from jax.experimental.pallas import tpu as pltpu
from jax.experimental.pallas import tpu_sc as plsc

N = 10000
C = 128
G = 8
RATIO = 0.8
E = 320000

LANE = 128
ROWS = 79
NPAD = ROWS * LANE

NW = 32
EPW = E // NW


@functools.cache
def _mesh():
    return plsc.VectorSubcoreMesh(
        core_axis_name="c", subcore_axis_name="s", num_cores=2,
        num_subcores=16)


def _sc_deg_body(col_hbm, ones_hbm, zeros_hbm, out_hbm, colv, ones_v, acc_sh):
    cid = lax.axis_index("c")
    sid = lax.axis_index("s")
    wid = cid * 16 + sid

    @pl.when(sid == 0)
    def _():
        pltpu.sync_copy(zeros_hbm, acc_sh)

    pltpu.sync_copy(col_hbm.at[wid], colv)
    pltpu.sync_copy(ones_hbm, ones_v)
    plsc.subcore_barrier()
    pltpu.sync_copy(ones_v, acc_sh.at[colv], add=True)
    plsc.subcore_barrier()

    @pl.when(sid == 0)
    def _():
        pltpu.sync_copy(acc_sh, out_hbm.at[cid])


@functools.cache
def _sc_deg_call():
    return pl.kernel(
        _sc_deg_body,
        out_type=jax.ShapeDtypeStruct((2, NPAD), jnp.float32),
        mesh=_mesh(),
        scratch_types=[
            pltpu.VMEM((EPW,), jnp.int32),
            pltpu.VMEM((EPW,), jnp.float32),
            pltpu.VMEM_SHARED((NPAD,), jnp.float32),
        ],
    )


def _sc_deg(col2, ones_e, zeros_n):
    return _sc_deg_call()(col2, ones_e, zeros_n)


def _sc_agg_body(row_hbm, col_hbm, u_hbm, zeros_hbm, out_hbm,
                 rowv, colv, vals, u_sh, acc_sh):
    cid = lax.axis_index("c")
    sid = lax.axis_index("s")
    wid = cid * 16 + sid

    @pl.when(sid == 0)
    def _():
        pltpu.sync_copy(u_hbm, u_sh)

    @pl.when(sid == 1)
    def _():
        pltpu.sync_copy(zeros_hbm, acc_sh)

    pltpu.sync_copy(row_hbm.at[wid], rowv)
    pltpu.sync_copy(col_hbm.at[wid], colv)
    plsc.subcore_barrier()
    pltpu.sync_copy(u_sh.at[rowv], vals)
    pltpu.sync_copy(vals, acc_sh.at[colv], add=True)
    plsc.subcore_barrier()

    @pl.when(sid == 0)
    def _():
        pltpu.sync_copy(acc_sh, out_hbm.at[cid])


@functools.cache
def _sc_agg_call():
    return pl.kernel(
        _sc_agg_body,
        out_type=jax.ShapeDtypeStruct((2, NPAD), jnp.float32),
        mesh=_mesh(),
        scratch_types=[
            pltpu.VMEM((EPW,), jnp.int32),
            pltpu.VMEM((EPW,), jnp.int32),
            pltpu.VMEM((EPW,), jnp.float32),
            pltpu.VMEM_SHARED((NPAD,), jnp.float32),
            pltpu.VMEM_SHARED((NPAD,), jnp.float32),
        ],
    )


def _sc_agg(row2, col2, u, zeros_n):
    return _sc_agg_call()(row2, col2, u, zeros_n)


def _mv_body(x_ref, w_ref, degp_ref, u_ref, dinv_ref):
    ii = lax.broadcasted_iota(jnp.int32, (LANE, LANE), 0)
    jj = lax.broadcasted_iota(jnp.int32, (LANE, LANE), 1)
    eye = jnp.where(ii == jj, 1.0, 0.0)
    ones_row = jnp.ones((1, LANE), jnp.float32)

    h_col = jnp.dot(x_ref[...], w_ref[...],
                    preferred_element_type=jnp.float32)
    h_row = jnp.dot(ones_row,
                    jnp.where(eye > 0.5,
                              jnp.broadcast_to(h_col, (LANE, LANE)), 0.0),
                    preferred_element_type=jnp.float32)
    deg = degp_ref[0, 0] + degp_ref[1, 0] + 1.0
    dinv = 1.0 / jnp.sqrt(deg)
    u_ref[0] = h_row * dinv
    dinv_ref[0] = dinv


_tc_mv = pl.pallas_call(
    _mv_body,
    grid=(ROWS,),
    in_specs=[
        pl.BlockSpec((LANE, C), lambda j: (j, 0)),
        pl.BlockSpec((C, 1), lambda j: (0, 0)),
        pl.BlockSpec((2, 1, 1, LANE), lambda j: (0, j, 0, 0)),
    ],
    out_specs=(
        pl.BlockSpec((1, 1, LANE), lambda j: (j, 0, 0)),
        pl.BlockSpec((1, 1, LANE), lambda j: (j, 0, 0)),
    ),
    out_shape=(
        jax.ShapeDtypeStruct((ROWS, 1, LANE), jnp.float32),
        jax.ShapeDtypeStruct((ROWS, 1, LANE), jnp.float32),
    ),
)


def _selpool_body(aggp_ref, u_ref, dinv_ref, batch_ref, b_ref, x_ref,
                  score_ref, pooled_ref, wsel_s, selm_s, vmax, vsum, vcnt):
    j = pl.program_id(0)
    neg = jnp.float32(-jnp.inf)
    ii = lax.broadcasted_iota(jnp.int32, (LANE, LANE), 0)
    jj = lax.broadcasted_iota(jnp.int32, (LANE, LANE), 1)
    eye = jnp.where(ii == jj, 1.0, 0.0)
    ones_col = jnp.ones((LANE, 1), jnp.float32)

    @pl.when(j == 0)
    def _():
        u = u_ref[...]
        dinv = dinv_ref[...]
        agg = aggp_ref[0] + aggp_ref[1]
        score = dinv * (agg + u) + b_ref[0, 0]
        score_ref[...] = score
        batch = batch_ref[...]

        MINI = jnp.int32(-2147483648)
        bits = lax.bitcast_convert_type(score, jnp.int32)
        ubits = jnp.where(bits >= 0, bits ^ MINI, ~bits)
        s_signed = ubits ^ MINI
        w = jnp.tanh(score)

        upper_incl = (ii <= jj).astype(jnp.float32)
        ri = lax.broadcasted_iota(jnp.int32, (ROWS, ROWS), 0)
        rj = lax.broadcasted_iota(jnp.int32, (ROWS, ROWS), 1)
        lower_strict = (rj < ri).astype(jnp.float32)

        wsel = jnp.zeros((ROWS, LANE), jnp.float32)
        selm = jnp.zeros((ROWS, LANE), jnp.float32)
        for g in range(G):
            ing = batch == g
            n_g = jnp.sum(jnp.where(ing, 1.0, 0.0))
            k_g = jnp.ceil(jnp.float32(RATIO) * n_g)

            def step(i, carry, ing=ing, k_g=k_g):
                P, a = carry
                bpos = 31 - i
                hi = lax.shift_right_logical(P, bpos) | 1
                eq = lax.shift_right_logical(ubits, bpos) == hi
                c1 = jnp.sum(jnp.where(eq & ing, 1.0, 0.0))
                take = (a + c1) >= k_g
                P2 = jnp.where(take, P | lax.shift_left(jnp.int32(1), bpos), P)
                a2 = jnp.where(take, a, a + c1)
                return (P2, a2)

            P, a = lax.fori_loop(0, 32, step, (jnp.int32(0), jnp.float32(0.0)))
            m_g = k_g - a
            strict = (s_signed > (P ^ MINI)) & ing
            ties = (ubits == P) & ing
            t = jnp.where(ties, 1.0, 0.0)
            incl = jnp.dot(t, upper_incl, preferred_element_type=jnp.float32)
            srow = incl[:, LANE - 1:LANE]
            rowpre = jnp.dot(lower_strict, srow,
                             preferred_element_type=jnp.float32)
            excl = rowpre + incl - t
            sel = strict | (ties & (excl < m_g))
            wsel = wsel + jnp.where(sel, w, 0.0)
            selm = selm + jnp.where(sel, 1.0, 0.0)
        wsel_s[...] = wsel
        selm_s[...] = selm
        vmax[...] = jnp.full((G, LANE), neg, jnp.float32)
        vsum[...] = jnp.zeros((G, LANE), jnp.float32)
        vcnt[...] = jnp.zeros((G, LANE), jnp.float32)

    def tocol(row):
        m = jnp.where(eye > 0.5, jnp.broadcast_to(row, (LANE, LANE)), 0.0)
        return jnp.dot(m, ones_col, preferred_element_type=jnp.float32)

    w_col = tocol(wsel_s[pl.ds(j, 1), :])
    s_col = tocol(selm_s[pl.ds(j, 1), :])
    b_col = tocol(batch_ref[pl.ds(j, 1), :].astype(jnp.float32))
    val = x_ref[...] * w_col
    for g in range(G):
        m = (s_col > 0.5) & (jnp.abs(b_col - g) < 0.5)
        gmax = jnp.max(jnp.where(m, val, neg), axis=0, keepdims=True)
        gsum = jnp.sum(jnp.where(m, val, 0.0), axis=0, keepdims=True)
        gcnt = jnp.sum(jnp.where(m, 1.0, 0.0))
        vmax[g:g + 1, :] = jnp.maximum(vmax[g:g + 1, :], gmax)
        vsum[g:g + 1, :] = vsum[g:g + 1, :] + gsum
        vcnt[g:g + 1, :] = vcnt[g:g + 1, :] + gcnt

    @pl.when(j == ROWS - 1)
    def _():
        pooled_ref[:, 0:LANE] = vmax[...]
        pooled_ref[:, LANE:2 * LANE] = (vsum[...]
                                        / jnp.maximum(vcnt[...], 1.0))


_tc_selpool = pl.pallas_call(
    _selpool_body,
    grid=(ROWS,),
    in_specs=[
        pl.BlockSpec((2, ROWS, LANE), lambda j: (0, 0, 0)),
        pl.BlockSpec((ROWS, LANE), lambda j: (0, 0)),
        pl.BlockSpec((ROWS, LANE), lambda j: (0, 0)),
        pl.BlockSpec((ROWS, LANE), lambda j: (0, 0)),
        pl.BlockSpec((1, 1), lambda j: (0, 0)),
        pl.BlockSpec((LANE, C), lambda j: (j, 0)),
    ],
    out_specs=(
        pl.BlockSpec((ROWS, LANE), lambda j: (0, 0)),
        pl.BlockSpec((G, 2 * C), lambda j: (0, 0)),
    ),
    out_shape=(
        jax.ShapeDtypeStruct((ROWS, LANE), jnp.float32),
        jax.ShapeDtypeStruct((G, 2 * C), jnp.float32),
    ),
    scratch_shapes=[
        pltpu.VMEM((ROWS, LANE), jnp.float32),
        pltpu.VMEM((ROWS, LANE), jnp.float32),
        pltpu.VMEM((G, LANE), jnp.float32),
        pltpu.VMEM((G, LANE), jnp.float32),
        pltpu.VMEM((G, LANE), jnp.float32),
    ],
)


def kernel(x, edge_index, batch, W, b):
    row2 = edge_index[0].reshape(NW, EPW)
    col2 = edge_index[1].reshape(NW, EPW)
    batch_pad = jnp.pad(batch, (0, NPAD - N), constant_values=G)
    zeros_n = jnp.zeros((NPAD,), jnp.float32)
    ones_e = jnp.ones((EPW,), jnp.float32)

    degp = _sc_deg(col2, ones_e, zeros_n)
    u3, dinv3 = _tc_mv(x, W, degp.reshape(2, ROWS, 1, LANE))
    aggp = _sc_agg(row2, col2, u3.reshape(NPAD), zeros_n)
    score79, pooled = _tc_selpool(
        aggp.reshape(2, ROWS, LANE), u3.reshape(ROWS, LANE),
        dinv3.reshape(ROWS, LANE), batch_pad.reshape(ROWS, LANE),
        b.reshape(1, 1), x)
    score = score79.reshape(NPAD)[:N]
    return (pooled, score)


# speedup vs baseline: 40.5899x; 1.0330x over previous
import functools

import jax
import jax.numpy as jnp
from jax import lax
from jax.experimental import pallas as pl
from jax.experimental.pallas import tpu as pltpu
from jax.experimental.pallas import tpu_sc as plsc

N = 10000
C = 128
G = 8
RATIO = 0.8
E = 320000

LANE = 128
ROWS = 80
NPAD = ROWS * LANE
NSL = NPAD // 16

NW = 32
EPW = E // NW
XBLK = 78


@functools.cache
def _mesh():
    return plsc.VectorSubcoreMesh(
        core_axis_name="c", subcore_axis_name="s", num_cores=2,
        num_subcores=16)


def _sc_deg_body(col_hbm, ones_hbm, zeros_hbm, out_hbm, colv, ones_v, acc_sh):
    cid = lax.axis_index("c")
    sid = lax.axis_index("s")
    wid = cid * 16 + sid

    @pl.when(sid == 0)
    def _():
        pltpu.sync_copy(zeros_hbm, acc_sh)

    pltpu.sync_copy(col_hbm.at[wid], colv)
    pltpu.sync_copy(ones_hbm, ones_v)
    plsc.subcore_barrier()
    pltpu.sync_copy(ones_v, acc_sh.at[colv], add=True)
    plsc.subcore_barrier()

    @pl.when(sid == 0)
    def _():
        pltpu.sync_copy(acc_sh, out_hbm.at[cid])


@functools.cache
def _sc_deg_call():
    return pl.kernel(
        _sc_deg_body,
        out_type=jax.ShapeDtypeStruct((2, NPAD), jnp.float32),
        mesh=_mesh(),
        scratch_types=[
            pltpu.VMEM((EPW,), jnp.int32),
            pltpu.VMEM((EPW,), jnp.float32),
            pltpu.VMEM_SHARED((NPAD,), jnp.float32),
        ],
    )


def _sc_deg(col2, ones_e, zeros_n):
    return _sc_deg_call()(col2, ones_e, zeros_n)


def _sc_agg_body(row_hbm, col_hbm, h_hbm, degp_hbm, zeros_hbm,
                 agg_out, u_out, dinv_out,
                 rowv, colv, vals, d0v, d1v, hv, uv, dinvv, aggv, sv,
                 u_sh, acc_sh):
    cid = lax.axis_index("c")
    sid = lax.axis_index("s")
    wid = cid * 16 + sid
    off = sid * NSL
    sl = pl.ds(off, NSL)

    pltpu.sync_copy(zeros_hbm.at[sl], acc_sh.at[sl])
    pltpu.sync_copy(row_hbm.at[wid], rowv)
    pltpu.sync_copy(col_hbm.at[wid], colv)
    pltpu.sync_copy(degp_hbm.at[0].at[sl], d0v)
    pltpu.sync_copy(degp_hbm.at[1].at[sl], d1v)
    pltpu.sync_copy(h_hbm.at[sl], hv)

    def nbody(i, carry):
        ix = pl.ds(i * 16, 16)
        d = d0v[ix] + d1v[ix] + 1.0
        bits = lax.bitcast_convert_type(d, jnp.int32)
        y = lax.bitcast_convert_type(
            jnp.int32(0x5F3759DF) - lax.shift_right_logical(bits, 1),
            jnp.float32)
        y = y * (1.5 - 0.5 * d * y * y)
        y = y * (1.5 - 0.5 * d * y * y)
        y = y * (1.5 - 0.5 * d * y * y)
        uv[ix] = hv[ix] * y
        dinvv[ix] = y
        return carry

    lax.fori_loop(0, NSL // 16, nbody, 0)
    pltpu.sync_copy(uv, u_sh.at[sl])

    @pl.when(cid == 0)
    def _():
        pltpu.sync_copy(uv, u_out.at[sl])
        pltpu.sync_copy(dinvv, dinv_out.at[sl])

    plsc.subcore_barrier()
    pltpu.sync_copy(u_sh.at[rowv], vals)
    pltpu.sync_copy(vals, acc_sh.at[colv], add=True)
    plsc.subcore_barrier()

    @pl.when(sid == 0)
    def _():
        pltpu.sync_copy(acc_sh, agg_out.at[cid])


@functools.cache
def _sc_agg_call():
    return pl.kernel(
        _sc_agg_body,
        out_type=(
            jax.ShapeDtypeStruct((2, NPAD), jnp.float32),
            jax.ShapeDtypeStruct((NPAD,), jnp.float32),
            jax.ShapeDtypeStruct((NPAD,), jnp.float32),
        ),
        mesh=_mesh(),
        scratch_types=[
            pltpu.VMEM((EPW,), jnp.int32),
            pltpu.VMEM((EPW,), jnp.int32),
            pltpu.VMEM((EPW,), jnp.float32),
            pltpu.VMEM((NSL,), jnp.float32),
            pltpu.VMEM((NSL,), jnp.float32),
            pltpu.VMEM((NSL,), jnp.float32),
            pltpu.VMEM((NSL,), jnp.float32),
            pltpu.VMEM((NSL,), jnp.float32),
            pltpu.VMEM((NSL,), jnp.float32),
            pltpu.VMEM((NSL,), jnp.float32),
            pltpu.VMEM_SHARED((NPAD,), jnp.float32),
            pltpu.VMEM_SHARED((NPAD,), jnp.float32),
        ],
    )


def _sc_agg(row2, col2, h, degp, zeros_n):
    return _sc_agg_call()(row2, col2, h, degp, zeros_n)


def _mv_body(x_ref, w_ref, h_ref):
    h_ref[...] = jnp.dot(x_ref[...], w_ref[...],
                         preferred_element_type=jnp.float32)


_tc_mv = pl.pallas_call(
    _mv_body,
    grid=(ROWS,),
    in_specs=[
        pl.BlockSpec((LANE, C), lambda j: (jnp.minimum(j, XBLK), 0)),
        pl.BlockSpec((C, 1), lambda j: (0, 0)),
    ],
    out_specs=pl.BlockSpec((LANE, 1), lambda j: (j, 0)),
    out_shape=jax.ShapeDtypeStruct((NPAD, 1), jnp.float32),
)


def _selpool_body(aggp_ref, u_ref, dinv_ref, batch_ref, b_ref, x_ref,
                  score_ref, pooled_ref, wsel_s, selm_s, vmax, vsum, vcnt):
    j = pl.program_id(0)
    neg = jnp.float32(-jnp.inf)
    ii = lax.broadcasted_iota(jnp.int32, (LANE, LANE), 0)
    jj = lax.broadcasted_iota(jnp.int32, (LANE, LANE), 1)
    eye = jnp.where(ii == jj, 1.0, 0.0)
    ones_col = jnp.ones((LANE, 1), jnp.float32)

    @pl.when(j == 0)
    def _():
        u = u_ref[...]
        dinv = dinv_ref[...]
        agg = aggp_ref[0] + aggp_ref[1]
        score = dinv * (agg + u) + b_ref[0, 0]
        score_ref[...] = score
        batch = batch_ref[...]

        MINI = jnp.int32(-2147483648)
        bits = lax.bitcast_convert_type(score, jnp.int32)
        ubits = jnp.where(bits >= 0, bits ^ MINI, ~bits)
        s_signed = ubits ^ MINI
        w = jnp.tanh(score)

        upper_incl = (ii <= jj).astype(jnp.float32)
        ri = lax.broadcasted_iota(jnp.int32, (ROWS, ROWS), 0)
        rj = lax.broadcasted_iota(jnp.int32, (ROWS, ROWS), 1)
        lower_strict = (rj < ri).astype(jnp.float32)

        wsel = jnp.zeros((ROWS, LANE), jnp.float32)
        selm = jnp.zeros((ROWS, LANE), jnp.float32)
        for g in range(G):
            ing = batch == g
            n_g = jnp.sum(jnp.where(ing, 1.0, 0.0))
            k_g = jnp.ceil(jnp.float32(RATIO) * n_g)

            def step(i, carry, ing=ing, k_g=k_g):
                P, a = carry
                bpos = 31 - i
                hi = lax.shift_right_logical(P, bpos) | 1
                eq = lax.shift_right_logical(ubits, bpos) == hi
                c1 = jnp.sum(jnp.where(eq & ing, 1.0, 0.0))
                take = (a + c1) >= k_g
                P2 = jnp.where(take, P | lax.shift_left(jnp.int32(1), bpos), P)
                a2 = jnp.where(take, a, a + c1)
                return (P2, a2)

            P, a = lax.fori_loop(0, 32, step, (jnp.int32(0), jnp.float32(0.0)))
            m_g = k_g - a
            strict = (s_signed > (P ^ MINI)) & ing
            ties = (ubits == P) & ing
            t = jnp.where(ties, 1.0, 0.0)
            incl = jnp.dot(t, upper_incl, preferred_element_type=jnp.float32)
            srow = incl[:, LANE - 1:LANE]
            rowpre = jnp.dot(lower_strict, srow,
                             preferred_element_type=jnp.float32)
            excl = rowpre + incl - t
            sel = strict | (ties & (excl < m_g))
            wsel = wsel + jnp.where(sel, w, 0.0)
            selm = selm + jnp.where(sel, 1.0, 0.0)
        wsel_s[...] = wsel
        selm_s[...] = selm
        vmax[...] = jnp.full((G, LANE), neg, jnp.float32)
        vsum[...] = jnp.zeros((G, LANE), jnp.float32)
        vcnt[...] = jnp.zeros((G, LANE), jnp.float32)

    def tocol(row):
        m = jnp.where(eye > 0.5, jnp.broadcast_to(row, (LANE, LANE)), 0.0)
        return jnp.dot(m, ones_col, preferred_element_type=jnp.float32)

    w_col = tocol(wsel_s[pl.ds(j, 1), :])
    s_col = tocol(selm_s[pl.ds(j, 1), :])
    b_col = tocol(batch_ref[pl.ds(j, 1), :].astype(jnp.float32))
    val = x_ref[...] * w_col
    for g in range(G):
        m = (s_col > 0.5) & (jnp.abs(b_col - g) < 0.5)
        gmax = jnp.max(jnp.where(m, val, neg), axis=0, keepdims=True)
        gsum = jnp.sum(jnp.where(m, val, 0.0), axis=0, keepdims=True)
        gcnt = jnp.sum(jnp.where(m, 1.0, 0.0))
        vmax[g:g + 1, :] = jnp.maximum(vmax[g:g + 1, :], gmax)
        vsum[g:g + 1, :] = vsum[g:g + 1, :] + gsum
        vcnt[g:g + 1, :] = vcnt[g:g + 1, :] + gcnt

    @pl.when(j == ROWS - 1)
    def _():
        pooled_ref[:, 0:LANE] = vmax[...]
        pooled_ref[:, LANE:2 * LANE] = (vsum[...]
                                        / jnp.maximum(vcnt[...], 1.0))


_tc_selpool = pl.pallas_call(
    _selpool_body,
    grid=(ROWS,),
    in_specs=[
        pl.BlockSpec((2, ROWS, LANE), lambda j: (0, 0, 0)),
        pl.BlockSpec((ROWS, LANE), lambda j: (0, 0)),
        pl.BlockSpec((ROWS, LANE), lambda j: (0, 0)),
        pl.BlockSpec((ROWS, LANE), lambda j: (0, 0)),
        pl.BlockSpec((1, 1), lambda j: (0, 0)),
        pl.BlockSpec((LANE, C), lambda j: (jnp.minimum(j, XBLK), 0)),
    ],
    out_specs=(
        pl.BlockSpec((ROWS, LANE), lambda j: (0, 0)),
        pl.BlockSpec((G, 2 * C), lambda j: (0, 0)),
    ),
    out_shape=(
        jax.ShapeDtypeStruct((ROWS, LANE), jnp.float32),
        jax.ShapeDtypeStruct((G, 2 * C), jnp.float32),
    ),
    scratch_shapes=[
        pltpu.VMEM((ROWS, LANE), jnp.float32),
        pltpu.VMEM((ROWS, LANE), jnp.float32),
        pltpu.VMEM((G, LANE), jnp.float32),
        pltpu.VMEM((G, LANE), jnp.float32),
        pltpu.VMEM((G, LANE), jnp.float32),
    ],
)


def kernel(x, edge_index, batch, W, b):
    row2 = edge_index[0].reshape(NW, EPW)
    col2 = edge_index[1].reshape(NW, EPW)
    batch_pad = jnp.pad(batch, (0, NPAD - N), constant_values=G)
    zeros_n = jnp.zeros((NPAD,), jnp.float32)
    ones_e = jnp.ones((EPW,), jnp.float32)

    h = _tc_mv(x, W)
    degp = _sc_deg(col2, ones_e, zeros_n)
    aggp, u, dinv = _sc_agg(row2, col2, h.reshape(NPAD), degp, zeros_n)
    score80, pooled = _tc_selpool(
        aggp.reshape(2, ROWS, LANE), u.reshape(ROWS, LANE),
        dinv.reshape(ROWS, LANE), batch_pad.reshape(ROWS, LANE),
        b.reshape(1, 1), x)
    score = score80.reshape(NPAD)[:N]
    return (pooled, score)


# speedup vs baseline: 88.0327x; 2.1688x over previous
import functools

import jax
import jax.numpy as jnp
from jax import lax
from jax.experimental import pallas as pl
from jax.experimental.pallas import tpu as pltpu
from jax.experimental.pallas import tpu_sc as plsc

N = 10000
C = 128
G = 8
RATIO = 0.8
E = 320000

LANE = 128
ROWS = 80
NPAD = ROWS * LANE
NSL = NPAD // 16

NW = 32
EPW = E // NW
XBLK = 78


@functools.cache
def _mesh():
    return plsc.VectorSubcoreMesh(
        core_axis_name="c", subcore_axis_name="s", num_cores=2,
        num_subcores=16)


def _sc_deg_body(col_hbm, ones_hbm, zeros_hbm, out_hbm, colv, ones_v, acc_sh):
    cid = lax.axis_index("c")
    sid = lax.axis_index("s")
    wid = cid * 16 + sid

    @pl.when(sid == 0)
    def _():
        pltpu.sync_copy(zeros_hbm, acc_sh)

    pltpu.sync_copy(col_hbm.at[wid], colv)
    pltpu.sync_copy(ones_hbm, ones_v)
    plsc.subcore_barrier()
    pltpu.sync_copy(ones_v, acc_sh.at[colv], add=True)
    plsc.subcore_barrier()

    @pl.when(sid == 0)
    def _():
        pltpu.sync_copy(acc_sh, out_hbm.at[cid])


@functools.cache
def _sc_deg_call():
    return pl.kernel(
        _sc_deg_body,
        out_type=jax.ShapeDtypeStruct((2, NPAD), jnp.float32),
        mesh=_mesh(),
        scratch_types=[
            pltpu.VMEM((EPW,), jnp.int32),
            pltpu.VMEM((EPW,), jnp.float32),
            pltpu.VMEM_SHARED((NPAD,), jnp.float32),
        ],
    )


def _sc_deg(col2, ones_e, zeros_n):
    return _sc_deg_call()(col2, ones_e, zeros_n)


def _sc_agg_body(row_hbm, col_hbm, h_hbm, degp_hbm, zeros_hbm,
                 agg_out, u_out, dinv_out,
                 rowv, colv, vals, d0v, d1v, hv, uv, dinvv, aggv, sv,
                 u_sh, acc_sh):
    cid = lax.axis_index("c")
    sid = lax.axis_index("s")
    wid = cid * 16 + sid
    off = sid * NSL
    sl = pl.ds(off, NSL)

    pltpu.sync_copy(zeros_hbm.at[sl], acc_sh.at[sl])
    pltpu.sync_copy(row_hbm.at[wid], rowv)
    pltpu.sync_copy(col_hbm.at[wid], colv)
    pltpu.sync_copy(degp_hbm.at[0].at[sl], d0v)
    pltpu.sync_copy(degp_hbm.at[1].at[sl], d1v)
    pltpu.sync_copy(h_hbm.at[sl], hv)

    def nbody(i, carry):
        ix = pl.ds(i * 16, 16)
        d = d0v[ix] + d1v[ix] + 1.0
        bits = lax.bitcast_convert_type(d, jnp.int32)
        y = lax.bitcast_convert_type(
            jnp.int32(0x5F3759DF) - lax.shift_right_logical(bits, 1),
            jnp.float32)
        y = y * (1.5 - 0.5 * d * y * y)
        y = y * (1.5 - 0.5 * d * y * y)
        y = y * (1.5 - 0.5 * d * y * y)
        uv[ix] = hv[ix] * y
        dinvv[ix] = y
        return carry

    lax.fori_loop(0, NSL // 16, nbody, 0)
    pltpu.sync_copy(uv, u_sh.at[sl])

    @pl.when(cid == 0)
    def _():
        pltpu.sync_copy(uv, u_out.at[sl])
        pltpu.sync_copy(dinvv, dinv_out.at[sl])

    plsc.subcore_barrier()
    pltpu.sync_copy(u_sh.at[rowv], vals)
    pltpu.sync_copy(vals, acc_sh.at[colv], add=True)
    plsc.subcore_barrier()

    @pl.when(sid == 0)
    def _():
        pltpu.sync_copy(acc_sh, agg_out.at[cid])


@functools.cache
def _sc_agg_call():
    return pl.kernel(
        _sc_agg_body,
        out_type=(
            jax.ShapeDtypeStruct((2, NPAD), jnp.float32),
            jax.ShapeDtypeStruct((NPAD,), jnp.float32),
            jax.ShapeDtypeStruct((NPAD,), jnp.float32),
        ),
        mesh=_mesh(),
        scratch_types=[
            pltpu.VMEM((EPW,), jnp.int32),
            pltpu.VMEM((EPW,), jnp.int32),
            pltpu.VMEM((EPW,), jnp.float32),
            pltpu.VMEM((NSL,), jnp.float32),
            pltpu.VMEM((NSL,), jnp.float32),
            pltpu.VMEM((NSL,), jnp.float32),
            pltpu.VMEM((NSL,), jnp.float32),
            pltpu.VMEM((NSL,), jnp.float32),
            pltpu.VMEM((NSL,), jnp.float32),
            pltpu.VMEM((NSL,), jnp.float32),
            pltpu.VMEM_SHARED((NPAD,), jnp.float32),
            pltpu.VMEM_SHARED((NPAD,), jnp.float32),
        ],
    )


def _sc_agg(row2, col2, h, degp, zeros_n):
    return _sc_agg_call()(row2, col2, h, degp, zeros_n)


def _mv_body(x_ref, w_ref, h_ref):
    h = jnp.dot(x_ref[...], w_ref[...], preferred_element_type=jnp.float32)
    h_ref[...] = jnp.concatenate(
        [h, jnp.zeros((NPAD - N, 1), jnp.float32)], axis=0)


_tc_mv = pl.pallas_call(
    _mv_body,
    out_shape=jax.ShapeDtypeStruct((NPAD, 1), jnp.float32),
)


SB = 4
RB = SB * LANE
NSTEP = ROWS // SB


def _selpool_body(aggp_ref, u_ref, dinv_ref, batch_ref, b_ref, x_ref,
                  score_ref, pooled_ref, wsel_s, selm_s, vmax, vsum, vcnt):
    j = pl.program_id(0)
    neg = jnp.float32(-jnp.inf)
    ii = lax.broadcasted_iota(jnp.int32, (LANE, LANE), 0)
    jj = lax.broadcasted_iota(jnp.int32, (LANE, LANE), 1)
    eye = jnp.where(ii == jj, 1.0, 0.0)
    ones_col = jnp.ones((LANE, 1), jnp.float32)

    @pl.when(j == 0)
    def _():
        u = u_ref[...]
        dinv = dinv_ref[...]
        agg = aggp_ref[0] + aggp_ref[1]
        score = dinv * (agg + u) + b_ref[0, 0]
        score_ref[...] = score
        batch = batch_ref[...]

        MINI = jnp.int32(-2147483648)
        bits = lax.bitcast_convert_type(score, jnp.int32)
        ubits = jnp.where(bits >= 0, bits ^ MINI, ~bits)
        s_signed = ubits ^ MINI
        w = jnp.tanh(score)

        ings = [batch == g for g in range(G)]
        ing_any = batch < G

        def msum(cond):
            return jnp.sum(jnp.where(cond, 1.0, 0.0))

        def pack8(vals):
            return jnp.concatenate(
                [v.reshape(1, 1) for v in vals], axis=1)

        def sel8(row8, cast=None):
            out = None
            for g in range(G):
                v = row8[0, g]
                term = jnp.where(ings[g], v, jnp.zeros_like(v))
                out = term if out is None else out + term
            return out

        n8 = pack8([msum(ings[g]) for g in range(G)])
        k8 = jnp.ceil(jnp.float32(RATIO) * n8)

        def step(i, carry):
            P8, a8 = carry
            bpos = 31 - i
            P_node = sel8(P8)
            eq = (lax.shift_right_logical(ubits, bpos)
                  == (lax.shift_right_logical(P_node, bpos) | 1))
            c18 = pack8([msum(eq & ings[g]) for g in range(G)])
            take = (a8 + c18) >= k8
            P2 = jnp.where(take, P8 | lax.shift_left(jnp.int32(1), bpos), P8)
            a2 = jnp.where(take, a8, a8 + c18)
            return (P2, a2)

        P8, a8 = lax.fori_loop(
            0, 32, step,
            (jnp.zeros((1, G), jnp.int32), jnp.zeros((1, G), jnp.float32)))
        m8 = k8 - a8
        T_node = sel8(P8)
        m_node = sel8(m8)
        strict = (s_signed > (T_node ^ MINI)) & ing_any
        ties = (ubits == T_node) & ing_any
        t = jnp.where(ties, 1.0, 0.0)
        upper_incl = (ii <= jj).astype(jnp.float32)
        ri = lax.broadcasted_iota(jnp.int32, (ROWS, ROWS), 0)
        rj = lax.broadcasted_iota(jnp.int32, (ROWS, ROWS), 1)
        lower_strict = (rj < ri).astype(jnp.float32)
        incl = jnp.dot(t, upper_incl, preferred_element_type=jnp.float32)
        srow = incl[:, LANE - 1:LANE]
        rowpre = jnp.dot(lower_strict, srow,
                         preferred_element_type=jnp.float32)
        excl = rowpre + incl - t
        pre8 = pack8([msum(ties & (batch < g)) for g in range(G)])
        rank = excl - sel8(pre8)
        sel = strict | (ties & (rank < m_node))
        wsel_s[...] = jnp.where(sel, w, 0.0)
        selm_s[...] = jnp.where(sel, 1.0, 0.0)
        vmax[...] = jnp.full((G, LANE), neg, jnp.float32)
        vsum[...] = jnp.zeros((G, LANE), jnp.float32)
        vcnt[...] = jnp.zeros((G, LANE), jnp.float32)

    def tocol(row):
        m = jnp.where(eye > 0.5, jnp.broadcast_to(row, (LANE, LANE)), 0.0)
        return jnp.dot(m, ones_col, preferred_element_type=jnp.float32)

    def tocols(rows):
        return jnp.concatenate(
            [tocol(rows[r:r + 1, :]) for r in range(SB)], axis=0)

    brows = batch_ref[pl.ds(j * SB, SB), :]
    w_col = tocols(wsel_s[pl.ds(j * SB, SB), :])
    s_col = tocols(selm_s[pl.ds(j * SB, SB), :])
    b_col = tocols(brows.astype(jnp.float32))
    val = x_ref[...] * w_col
    gmin = jnp.min(brows)
    gmax_b = jnp.minimum(jnp.max(brows), G - 1)

    def gbody(g, carry):
        gf = g.astype(jnp.float32)
        m = (s_col > 0.5) & (jnp.abs(b_col - gf) < 0.5)
        bmax = jnp.max(jnp.where(m, val, neg), axis=0, keepdims=True)
        bsum = jnp.sum(jnp.where(m, val, 0.0), axis=0, keepdims=True)
        bcnt = jnp.sum(jnp.where(m, 1.0, 0.0))
        gs = pl.ds(g, 1)
        vmax[gs, :] = jnp.maximum(vmax[gs, :], bmax)
        vsum[gs, :] = vsum[gs, :] + bsum
        vcnt[gs, :] = vcnt[gs, :] + bcnt
        return carry

    lax.fori_loop(gmin, gmax_b + 1, gbody, 0)

    @pl.when(j == NSTEP - 1)
    def _():
        pooled_ref[:, 0:LANE] = vmax[...]
        pooled_ref[:, LANE:2 * LANE] = (vsum[...]
                                        / jnp.maximum(vcnt[...], 1.0))


_tc_selpool = pl.pallas_call(
    _selpool_body,
    grid=(NSTEP,),
    in_specs=[
        pl.BlockSpec((2, ROWS, LANE), lambda j: (0, 0, 0)),
        pl.BlockSpec((ROWS, LANE), lambda j: (0, 0)),
        pl.BlockSpec((ROWS, LANE), lambda j: (0, 0)),
        pl.BlockSpec((ROWS, LANE), lambda j: (0, 0)),
        pl.BlockSpec((1, 1), lambda j: (0, 0)),
        pl.BlockSpec((RB, C), lambda j: (j, 0)),
    ],
    out_specs=(
        pl.BlockSpec((ROWS, LANE), lambda j: (0, 0)),
        pl.BlockSpec((G, 2 * C), lambda j: (0, 0)),
    ),
    out_shape=(
        jax.ShapeDtypeStruct((ROWS, LANE), jnp.float32),
        jax.ShapeDtypeStruct((G, 2 * C), jnp.float32),
    ),
    scratch_shapes=[
        pltpu.VMEM((ROWS, LANE), jnp.float32),
        pltpu.VMEM((ROWS, LANE), jnp.float32),
        pltpu.VMEM((G, LANE), jnp.float32),
        pltpu.VMEM((G, LANE), jnp.float32),
        pltpu.VMEM((G, LANE), jnp.float32),
    ],
)


def kernel(x, edge_index, batch, W, b):
    row2 = edge_index[0].reshape(NW, EPW)
    col2 = edge_index[1].reshape(NW, EPW)
    batch_pad = jnp.pad(batch, (0, NPAD - N), constant_values=G)
    zeros_n = jnp.zeros((NPAD,), jnp.float32)
    ones_e = jnp.ones((EPW,), jnp.float32)

    h = _tc_mv(x, W)
    degp = _sc_deg(col2, ones_e, zeros_n)
    aggp, u, dinv = _sc_agg(row2, col2, h.reshape(NPAD), degp, zeros_n)
    score80, pooled = _tc_selpool(
        aggp.reshape(2, ROWS, LANE), u.reshape(ROWS, LANE),
        dinv.reshape(ROWS, LANE), batch_pad.reshape(ROWS, LANE),
        b.reshape(1, 1), x)
    score = score80.reshape(NPAD)[:N]
    return (pooled, score)


# speedup vs baseline: 100.0660x; 1.1367x over previous
import functools

import jax
import jax.numpy as jnp
from jax import lax
from jax.experimental import pallas as pl
from jax.experimental.pallas import tpu as pltpu
from jax.experimental.pallas import tpu_sc as plsc

N = 10000
C = 128
G = 8
RATIO = 0.8
E = 320000

LANE = 128
ROWS = 80
NPAD = ROWS * LANE
NSL = NPAD // 16

NW = 32
EPW = E // NW
XBLK = 78


@functools.cache
def _mesh():
    return plsc.VectorSubcoreMesh(
        core_axis_name="c", subcore_axis_name="s", num_cores=2,
        num_subcores=16)


def _sc_deg_body(e_hbm, ones_hbm, zeros_hbm, out_hbm, colv, ones_v, acc_sh):
    cid = lax.axis_index("c")
    sid = lax.axis_index("s")
    wid = cid * 16 + sid

    @pl.when(sid == 0)
    def _():
        pltpu.sync_copy(zeros_hbm, acc_sh)

    pltpu.sync_copy(e_hbm.at[1].at[wid], colv)
    pltpu.sync_copy(ones_hbm, ones_v)
    plsc.subcore_barrier()
    pltpu.sync_copy(ones_v, acc_sh.at[colv], add=True)
    plsc.subcore_barrier()

    @pl.when(sid == 0)
    def _():
        pltpu.sync_copy(acc_sh, out_hbm.at[cid])


@functools.cache
def _sc_deg_call():
    return pl.kernel(
        _sc_deg_body,
        out_type=jax.ShapeDtypeStruct((2, NPAD), jnp.float32),
        mesh=_mesh(),
        scratch_types=[
            pltpu.VMEM((EPW,), jnp.int32),
            pltpu.VMEM((EPW,), jnp.float32),
            pltpu.VMEM_SHARED((NPAD,), jnp.float32),
        ],
    )


def _sc_deg(e3, ones_e, zeros_n):
    return _sc_deg_call()(e3, ones_e, zeros_n)


def _sc_agg_body(e_hbm, h_hbm, degp_hbm, zeros_hbm,
                 agg_out, u_out, dinv_out,
                 rowv, colv, vals, d0v, d1v, hv, uv, dinvv, aggv, sv,
                 u_sh, acc_sh):
    cid = lax.axis_index("c")
    sid = lax.axis_index("s")
    wid = cid * 16 + sid
    off = sid * NSL
    sl = pl.ds(off, NSL)

    pltpu.sync_copy(zeros_hbm.at[sl], acc_sh.at[sl])
    pltpu.sync_copy(e_hbm.at[0].at[wid], rowv)
    pltpu.sync_copy(e_hbm.at[1].at[wid], colv)
    pltpu.sync_copy(degp_hbm.at[0].at[sl], d0v)
    pltpu.sync_copy(degp_hbm.at[1].at[sl], d1v)
    pltpu.sync_copy(h_hbm.at[sl], hv)

    def nbody(i, carry):
        ix = pl.ds(i * 16, 16)
        d = d0v[ix] + d1v[ix] + 1.0
        bits = lax.bitcast_convert_type(d, jnp.int32)
        y = lax.bitcast_convert_type(
            jnp.int32(0x5F3759DF) - lax.shift_right_logical(bits, 1),
            jnp.float32)
        y = y * (1.5 - 0.5 * d * y * y)
        y = y * (1.5 - 0.5 * d * y * y)
        y = y * (1.5 - 0.5 * d * y * y)
        uv[ix] = hv[ix] * y
        dinvv[ix] = y
        return carry

    lax.fori_loop(0, NSL // 16, nbody, 0)
    pltpu.sync_copy(uv, u_sh.at[sl])

    @pl.when(cid == 0)
    def _():
        pltpu.sync_copy(uv, u_out.at[sl])
        pltpu.sync_copy(dinvv, dinv_out.at[sl])

    plsc.subcore_barrier()
    pltpu.sync_copy(u_sh.at[rowv], vals)
    pltpu.sync_copy(vals, acc_sh.at[colv], add=True)
    plsc.subcore_barrier()

    @pl.when(sid == 0)
    def _():
        pltpu.sync_copy(acc_sh, agg_out.at[cid])


@functools.cache
def _sc_agg_call():
    return pl.kernel(
        _sc_agg_body,
        out_type=(
            jax.ShapeDtypeStruct((2, NPAD), jnp.float32),
            jax.ShapeDtypeStruct((NPAD,), jnp.float32),
            jax.ShapeDtypeStruct((NPAD,), jnp.float32),
        ),
        mesh=_mesh(),
        scratch_types=[
            pltpu.VMEM((EPW,), jnp.int32),
            pltpu.VMEM((EPW,), jnp.int32),
            pltpu.VMEM((EPW,), jnp.float32),
            pltpu.VMEM((NSL,), jnp.float32),
            pltpu.VMEM((NSL,), jnp.float32),
            pltpu.VMEM((NSL,), jnp.float32),
            pltpu.VMEM((NSL,), jnp.float32),
            pltpu.VMEM((NSL,), jnp.float32),
            pltpu.VMEM((NSL,), jnp.float32),
            pltpu.VMEM((NSL,), jnp.float32),
            pltpu.VMEM_SHARED((NPAD,), jnp.float32),
            pltpu.VMEM_SHARED((NPAD,), jnp.float32),
        ],
    )


def _sc_agg(e3, h, degp, zeros_n):
    return _sc_agg_call()(e3, h, degp, zeros_n)


def _mv_body(x_ref, w_ref, h_ref):
    h = jnp.dot(x_ref[...], w_ref[...], preferred_element_type=jnp.float32)
    h = jnp.concatenate([h, jnp.zeros((NPAD - N, 1), jnp.float32)], axis=0)
    h_ref[...] = h.reshape(ROWS, LANE)


_tc_mv = pl.pallas_call(
    _mv_body,
    out_shape=jax.ShapeDtypeStruct((ROWS, LANE), jnp.float32),
)


SB = 4
RB = SB * LANE
NSTEP = ROWS // SB


def _selpool_body(aggp_ref, u_ref, dinv_ref, batch_ref, b_ref, x_ref,
                  score_ref, pooled_ref, wsel_s, selm_s, vmax, vsum, vcnt):
    j = pl.program_id(0)
    neg = jnp.float32(-jnp.inf)
    ii = lax.broadcasted_iota(jnp.int32, (LANE, LANE), 0)
    jj = lax.broadcasted_iota(jnp.int32, (LANE, LANE), 1)
    eye = jnp.where(ii == jj, 1.0, 0.0)
    ones_col = jnp.ones((LANE, 1), jnp.float32)

    @pl.when(j == 0)
    def _():
        u = u_ref[...]
        dinv = dinv_ref[...]
        agg = aggp_ref[0] + aggp_ref[1]
        score = dinv * (agg + u) + b_ref[0, 0]
        score_ref[...] = score
        batch = batch_ref[...]

        MINI = jnp.int32(-2147483648)
        bits = lax.bitcast_convert_type(score, jnp.int32)
        ubits = jnp.where(bits >= 0, bits ^ MINI, ~bits)
        s_signed = ubits ^ MINI
        w = jnp.tanh(score)

        ings = [batch == g for g in range(G)]
        ing_any = batch < G

        def msum(cond):
            return jnp.sum(jnp.where(cond, 1.0, 0.0))

        def pack8(vals):
            return jnp.concatenate(
                [v.reshape(1, 1) for v in vals], axis=1)

        def sel8(row8, cast=None):
            out = None
            for g in range(G):
                v = row8[0, g]
                term = jnp.where(ings[g], v, jnp.zeros_like(v))
                out = term if out is None else out + term
            return out

        n8 = pack8([msum(ings[g]) for g in range(G)])
        k8 = jnp.ceil(jnp.float32(RATIO) * n8)

        def step(i, carry):
            P8, a8 = carry
            bpos = 31 - i
            P_node = sel8(P8)
            eq = (lax.shift_right_logical(ubits, bpos)
                  == (lax.shift_right_logical(P_node, bpos) | 1))
            c18 = pack8([msum(eq & ings[g]) for g in range(G)])
            take = (a8 + c18) >= k8
            P2 = jnp.where(take, P8 | lax.shift_left(jnp.int32(1), bpos), P8)
            a2 = jnp.where(take, a8, a8 + c18)
            return (P2, a2)

        P8, a8 = lax.fori_loop(
            0, 32, step,
            (jnp.zeros((1, G), jnp.int32), jnp.zeros((1, G), jnp.float32)))
        m8 = k8 - a8
        T_node = sel8(P8)
        m_node = sel8(m8)
        strict = (s_signed > (T_node ^ MINI)) & ing_any
        ties = (ubits == T_node) & ing_any
        t = jnp.where(ties, 1.0, 0.0)
        upper_incl = (ii <= jj).astype(jnp.float32)
        ri = lax.broadcasted_iota(jnp.int32, (ROWS, ROWS), 0)
        rj = lax.broadcasted_iota(jnp.int32, (ROWS, ROWS), 1)
        lower_strict = (rj < ri).astype(jnp.float32)
        incl = jnp.dot(t, upper_incl, preferred_element_type=jnp.float32)
        srow = incl[:, LANE - 1:LANE]
        rowpre = jnp.dot(lower_strict, srow,
                         preferred_element_type=jnp.float32)
        excl = rowpre + incl - t
        pre8 = pack8([msum(ties & (batch < g)) for g in range(G)])
        rank = excl - sel8(pre8)
        sel = strict | (ties & (rank < m_node))
        wsel_s[...] = jnp.where(sel, w, 0.0)
        selm_s[...] = jnp.where(sel, 1.0, 0.0)
        vmax[...] = jnp.full((G, LANE), neg, jnp.float32)
        vsum[...] = jnp.zeros((G, LANE), jnp.float32)
        vcnt[...] = jnp.zeros((G, LANE), jnp.float32)

    def tocol(row):
        m = jnp.where(eye > 0.5, jnp.broadcast_to(row, (LANE, LANE)), 0.0)
        return jnp.dot(m, ones_col, preferred_element_type=jnp.float32)

    def tocols(rows):
        return jnp.concatenate(
            [tocol(rows[r:r + 1, :]) for r in range(SB)], axis=0)

    brows = batch_ref[pl.ds(j * SB, SB), :]
    w_col = tocols(wsel_s[pl.ds(j * SB, SB), :])
    s_col = tocols(selm_s[pl.ds(j * SB, SB), :])
    b_col = tocols(brows.astype(jnp.float32))
    val = x_ref[...] * w_col
    gmin = jnp.min(brows)
    gmax_b = jnp.minimum(jnp.max(brows), G - 1)

    def gbody(g, carry):
        gf = g.astype(jnp.float32)
        m = (s_col > 0.5) & (jnp.abs(b_col - gf) < 0.5)
        bmax = jnp.max(jnp.where(m, val, neg), axis=0, keepdims=True)
        bsum = jnp.sum(jnp.where(m, val, 0.0), axis=0, keepdims=True)
        bcnt = jnp.sum(jnp.where(m, 1.0, 0.0))
        gs = pl.ds(g, 1)
        vmax[gs, :] = jnp.maximum(vmax[gs, :], bmax)
        vsum[gs, :] = vsum[gs, :] + bsum
        vcnt[gs, :] = vcnt[gs, :] + bcnt
        return carry

    lax.fori_loop(gmin, gmax_b + 1, gbody, 0)

    @pl.when(j == NSTEP - 1)
    def _():
        pooled_ref[:, 0:LANE] = vmax[...]
        pooled_ref[:, LANE:2 * LANE] = (vsum[...]
                                        / jnp.maximum(vcnt[...], 1.0))


_tc_selpool = pl.pallas_call(
    _selpool_body,
    grid=(NSTEP,),
    in_specs=[
        pl.BlockSpec((2, ROWS, LANE), lambda j: (0, 0, 0)),
        pl.BlockSpec((ROWS, LANE), lambda j: (0, 0)),
        pl.BlockSpec((ROWS, LANE), lambda j: (0, 0)),
        pl.BlockSpec((ROWS, LANE), lambda j: (0, 0)),
        pl.BlockSpec((1, 1), lambda j: (0, 0)),
        pl.BlockSpec((RB, C), lambda j: (j, 0)),
    ],
    out_specs=(
        pl.BlockSpec((ROWS, LANE), lambda j: (0, 0)),
        pl.BlockSpec((G, 2 * C), lambda j: (0, 0)),
    ),
    out_shape=(
        jax.ShapeDtypeStruct((ROWS, LANE), jnp.float32),
        jax.ShapeDtypeStruct((G, 2 * C), jnp.float32),
    ),
    scratch_shapes=[
        pltpu.VMEM((ROWS, LANE), jnp.float32),
        pltpu.VMEM((ROWS, LANE), jnp.float32),
        pltpu.VMEM((G, LANE), jnp.float32),
        pltpu.VMEM((G, LANE), jnp.float32),
        pltpu.VMEM((G, LANE), jnp.float32),
    ],
)


def kernel(x, edge_index, batch, W, b):
    e3 = edge_index.reshape(2, NW, EPW)
    batch_pad = jnp.pad(batch, (0, NPAD - N), constant_values=G)
    zeros_n = jnp.zeros((NPAD,), jnp.float32)
    ones_e = jnp.ones((EPW,), jnp.float32)

    h = _tc_mv(x, W)
    degp = _sc_deg(e3, ones_e, zeros_n)
    aggp, u, dinv = _sc_agg(e3, h.reshape(NPAD), degp, zeros_n)
    score80, pooled = _tc_selpool(
        aggp.reshape(2, ROWS, LANE), u.reshape(ROWS, LANE),
        dinv.reshape(ROWS, LANE), batch_pad.reshape(ROWS, LANE),
        b.reshape(1, 1), x)
    score = score80.reshape(NPAD)[:N]
    return (pooled, score)
